# baseline (speedup 1.0000x reference)
import jax
import jax.numpy as jnp
from jax.experimental import pallas as pl
from jax.experimental.pallas import tpu as pltpu


def _fused_kernel(a_ref, x_ref, w0_ref, b0_ref, w1_ref, b1_ref, out_ref):
    BB, N, Nd = x_ref.shape
    d = Nd // N
    a = a_ref[...]
    xp = x_ref[...]

    x2 = jnp.concatenate(
        [xp[:, :, k * d:(k + 1) * d] for k in range(N)], axis=1)

    h = jnp.dot(x2.reshape(BB * N * N, d), w0_ref[...],
                preferred_element_type=jnp.float32)
    h = jnp.maximum(h + b0_ref[...], 0.0)
    x0 = h.reshape(BB, N, N, d)

    rowsum = jnp.sum(a, axis=2) * (1.0 / N)
    s = jnp.sum(x0 * rowsum[:, :, None, None], axis=1)
    diag = jnp.sum(x0 * a[:, :, :, None], axis=1)

    y = jnp.sum(x0, axis=2) * (1.0 / N)
    nctx = jax.lax.dot_general(a, y, (((1,), (1,)), ((0,), (0,))),
                               preferred_element_type=jnp.float32)

    w1 = w1_ref[...]
    p = (jnp.dot(s.reshape(BB * N, d), w1[0:d],
                 preferred_element_type=jnp.float32)
         + jnp.dot(diag.reshape(BB * N, d), w1[d:2 * d],
                   preferred_element_type=jnp.float32)
         + b1_ref[...])
    q = jnp.dot(nctx.reshape(BB * N, d), w1[2 * d:3 * d],
                preferred_element_type=jnp.float32)

    p3 = p.reshape(BB, N, d)
    q3 = q.reshape(BB, N, d)
    out_ref[...] = jnp.concatenate(
        [p3 + q3[:, j:j + 1, :] for j in range(N)], axis=2)


def kernel(A, X, W0, b0, W1, b1):
    B, N, _, d = X.shape
    outdim = W1.shape[1]
    BB = 32
    grid = (B // BB,)
    out = pl.pallas_call(
        _fused_kernel,
        grid=grid,
        in_specs=[
            pl.BlockSpec((BB, N, N), lambda b: (b, 0, 0)),
            pl.BlockSpec((BB, N, N * d), lambda b: (b, 0, 0)),
            pl.BlockSpec((d, d), lambda b: (0, 0)),
            pl.BlockSpec((1, d), lambda b: (0, 0)),
            pl.BlockSpec((3 * d, outdim), lambda b: (0, 0)),
            pl.BlockSpec((1, outdim), lambda b: (0, 0)),
        ],
        out_specs=pl.BlockSpec((BB, N, N * outdim), lambda b: (b, 0, 0)),
        out_shape=jax.ShapeDtypeStruct((B, N, N * outdim), jnp.float32),
        compiler_params=pltpu.CompilerParams(
            dimension_semantics=("parallel",)),
    )(A, X.reshape(B, N, N * d), W0, b0.reshape(1, d), W1,
      b1.reshape(1, outdim))
    return out.reshape(B, N, N, outdim)


# speedup vs baseline: 1.0975x; 1.0975x over previous
import jax
import jax.numpy as jnp
from jax.experimental import pallas as pl
from jax.experimental.pallas import tpu as pltpu


def _fused_kernel(a_ref, x_ref, w0_ref, b0_ref, w1_ref, b1_ref, out_ref):
    BB, N, Nd = x_ref.shape
    d = Nd // N
    f32 = jnp.float32
    a = a_ref[...]
    xp = x_ref[...]
    w0 = w0_ref[...]
    b0 = b0_ref[...]
    w1 = w1_ref[...]

    hs = []
    for k in range(N):
        xs = xp[:, :, k * d:(k + 1) * d].reshape(BB * N, d)
        hk = jnp.maximum(
            jnp.dot(xs, w0, preferred_element_type=f32) + b0, 0.0)
        hs.append(hk.reshape(BB, N, d))
    x0p = jnp.concatenate(hs, axis=2)

    lane_k = jax.lax.broadcasted_iota(jnp.int32, (N, Nd), 1) // d
    row_k = jax.lax.broadcasted_iota(jnp.int32, (N, Nd), 0)
    rl = (lane_k == row_k).astype(f32)

    rowsum = jnp.sum(a, axis=2) * (1.0 / N)
    rwexp = jnp.dot(rowsum, rl, preferred_element_type=f32)
    at = jnp.swapaxes(a, 1, 2)
    aexp = jnp.dot(at.reshape(BB * N, N), rl,
                   preferred_element_type=f32).reshape(BB, N, Nd)

    tw1s = jnp.concatenate([w1[0:d]] * N, axis=0)
    tw1d = jnp.concatenate([w1[d:2 * d]] * N, axis=0)
    ss = (x0p * rwexp[:, None, :]).reshape(BB * N, Nd)
    dd = (x0p * aexp).reshape(BB * N, Nd)
    p = (jnp.dot(ss, tw1s, preferred_element_type=f32)
         + jnp.dot(dd, tw1d, preferred_element_type=f32)
         + b1_ref[...])

    yp = jnp.sum(x0p, axis=1) * (1.0 / N)
    y3 = jnp.concatenate(
        [yp[:, None, k * d:(k + 1) * d] for k in range(N)], axis=1)
    nctx = jax.lax.dot_general(a, y3, (((1,), (1,)), ((0,), (0,))),
                               preferred_element_type=f32)
    q = jnp.dot(nctx.reshape(BB * N, d), w1[2 * d:3 * d],
                preferred_element_type=f32)

    lane_d = jax.lax.broadcasted_iota(jnp.int32, (d, Nd), 1) % d
    row_d = jax.lax.broadcasted_iota(jnp.int32, (d, Nd), 0)
    rt = (lane_d == row_d).astype(f32)
    ptile = jnp.dot(p, rt, preferred_element_type=f32).reshape(BB, N, Nd)
    q3 = q.reshape(BB, N, d)
    qpack = jnp.concatenate(
        [q3[:, j:j + 1, :] for j in range(N)], axis=2)
    out_ref[...] = ptile + qpack


def kernel(A, X, W0, b0, W1, b1):
    B, N, _, d = X.shape
    outdim = W1.shape[1]
    BB = 32
    grid = (B // BB,)
    out = pl.pallas_call(
        _fused_kernel,
        grid=grid,
        in_specs=[
            pl.BlockSpec((BB, N, N), lambda b: (b, 0, 0)),
            pl.BlockSpec((BB, N, N * d), lambda b: (b, 0, 0)),
            pl.BlockSpec((d, d), lambda b: (0, 0)),
            pl.BlockSpec((1, d), lambda b: (0, 0)),
            pl.BlockSpec((3 * d, outdim), lambda b: (0, 0)),
            pl.BlockSpec((1, outdim), lambda b: (0, 0)),
        ],
        out_specs=pl.BlockSpec((BB, N, N * outdim), lambda b: (b, 0, 0)),
        out_shape=jax.ShapeDtypeStruct((B, N, N * outdim), jnp.float32),
        compiler_params=pltpu.CompilerParams(
            dimension_semantics=("parallel",)),
    )(A, X.reshape(B, N, N * d), W0, b0.reshape(1, d), W1,
      b1.reshape(1, outdim))
    return out.reshape(B, N, N, outdim)


# speedup vs baseline: 1.2334x; 1.1239x over previous
import jax
import jax.numpy as jnp
from jax.experimental import pallas as pl
from jax.experimental.pallas import tpu as pltpu


def _fused_kernel(a_ref, x_ref, w0_ref, b0_ref, w1_ref, b1_ref, out_ref):
    BB, N, Nd = x_ref.shape
    d = Nd // N
    f32 = jnp.float32
    a = a_ref[...]
    xp = x_ref[...]
    w0 = w0_ref[...]
    b0 = b0_ref[...]
    w1 = w1_ref[...]

    zd = jnp.zeros((d, d), f32)
    w0two = jnp.concatenate(
        [jnp.concatenate([w0, zd], axis=1),
         jnp.concatenate([zd, w0], axis=1)], axis=0)
    hs = []
    for kk in range(N // 2):
        xs = xp[:, :, kk * 2 * d:(kk + 1) * 2 * d].reshape(BB * N, 2 * d)
        hs.append(jnp.dot(xs, w0two,
                          preferred_element_type=f32).reshape(BB, N, 2 * d))
    b0tile = jnp.concatenate([b0] * N, axis=1)
    x0p = jnp.maximum(jnp.concatenate(hs, axis=2) + b0tile[:, None, :], 0.0)

    lane_k = jax.lax.broadcasted_iota(jnp.int32, (N, Nd), 1) // d
    row_k = jax.lax.broadcasted_iota(jnp.int32, (N, Nd), 0)
    rl = (lane_k == row_k).astype(f32)

    rowsum = jnp.sum(a, axis=2) * (1.0 / N)
    rwexp = jnp.dot(rowsum, rl, preferred_element_type=f32)
    at = jnp.swapaxes(a, 1, 2)
    aexp = jnp.dot(at.reshape(BB * N, N), rl,
                   preferred_element_type=f32).reshape(BB, N, Nd)

    tw1s = jnp.concatenate([w1[0:d]] * N, axis=0)
    tw1d = jnp.concatenate([w1[d:2 * d]] * N, axis=0)
    ss = (x0p * rwexp[:, None, :]).reshape(BB * N, Nd)
    dd = (x0p * aexp).reshape(BB * N, Nd)
    p = (jnp.dot(ss, tw1s, preferred_element_type=f32)
         + jnp.dot(dd, tw1d, preferred_element_type=f32)
         + b1_ref[...])

    yp = jnp.sum(x0p, axis=1) * (1.0 / N)
    y3 = jnp.concatenate(
        [yp[:, None, k * d:(k + 1) * d] for k in range(N)], axis=1)
    nctx = jax.lax.dot_general(a, y3, (((1,), (1,)), ((0,), (0,))),
                               preferred_element_type=f32)
    q = jnp.dot(nctx.reshape(BB * N, d), w1[2 * d:3 * d],
                preferred_element_type=f32)

    lane_d = jax.lax.broadcasted_iota(jnp.int32, (d, Nd), 1) % d
    row_d = jax.lax.broadcasted_iota(jnp.int32, (d, Nd), 0)
    rt = (lane_d == row_d).astype(f32)
    ptile = jnp.dot(p, rt, preferred_element_type=f32).reshape(BB, N, Nd)
    q3 = q.reshape(BB, N, d)
    qpack = jnp.concatenate(
        [q3[:, j:j + 1, :] for j in range(N)], axis=2)
    out_ref[...] = ptile + qpack


def kernel(A, X, W0, b0, W1, b1):
    B, N, _, d = X.shape
    outdim = W1.shape[1]
    BB = 32
    grid = (B // BB,)
    out = pl.pallas_call(
        _fused_kernel,
        grid=grid,
        in_specs=[
            pl.BlockSpec((BB, N, N), lambda b: (b, 0, 0)),
            pl.BlockSpec((BB, N, N * d), lambda b: (b, 0, 0)),
            pl.BlockSpec((d, d), lambda b: (0, 0)),
            pl.BlockSpec((1, d), lambda b: (0, 0)),
            pl.BlockSpec((3 * d, outdim), lambda b: (0, 0)),
            pl.BlockSpec((1, outdim), lambda b: (0, 0)),
        ],
        out_specs=pl.BlockSpec((BB, N, N * outdim), lambda b: (b, 0, 0)),
        out_shape=jax.ShapeDtypeStruct((B, N, N * outdim), jnp.float32),
        compiler_params=pltpu.CompilerParams(
            dimension_semantics=("parallel",)),
    )(A, X.reshape(B, N, N * d), W0, b0.reshape(1, d), W1,
      b1.reshape(1, outdim))
    return out.reshape(B, N, N, outdim)


# speedup vs baseline: 1.2484x; 1.0121x over previous
import jax
import jax.numpy as jnp
from jax.experimental import pallas as pl
from jax.experimental.pallas import tpu as pltpu


def _fused_kernel(a_ref, x_ref, w0_ref, b0_ref, w1_ref, b1_ref, out_ref):
    BB, N, Nd = x_ref.shape
    d = Nd // N
    f32 = jnp.float32
    a = a_ref[...]
    xp = x_ref[...]
    w0 = w0_ref[...]
    b0 = b0_ref[...]
    w1 = w1_ref[...]

    zd = jnp.zeros((d, d), f32)
    w0two = jnp.concatenate(
        [jnp.concatenate([w0, zd], axis=1),
         jnp.concatenate([zd, w0], axis=1)], axis=0)
    w0two16 = w0two.astype(jnp.bfloat16)
    hs = []
    for kk in range(N // 2):
        xs = xp[:, :, kk * 2 * d:(kk + 1) * 2 * d].reshape(BB * N, 2 * d)
        hs.append(jnp.dot(xs.astype(jnp.bfloat16), w0two16,
                          preferred_element_type=f32).reshape(BB, N, 2 * d))
    b0tile = jnp.concatenate([b0] * N, axis=1)
    x0p = jnp.maximum(jnp.concatenate(hs, axis=2) + b0tile[:, None, :], 0.0)

    lane_k = jax.lax.broadcasted_iota(jnp.int32, (N, Nd), 1) // d
    row_k = jax.lax.broadcasted_iota(jnp.int32, (N, Nd), 0)
    rl = (lane_k == row_k).astype(f32)

    rowsum = jnp.sum(a, axis=2) * (1.0 / N)
    rwexp = jnp.dot(rowsum, rl, preferred_element_type=f32)
    at = jnp.swapaxes(a, 1, 2)
    aexp = jnp.dot(at.reshape(BB * N, N), rl,
                   preferred_element_type=f32).reshape(BB, N, Nd)

    tw1s = jnp.concatenate([w1[0:d]] * N, axis=0)
    tw1d = jnp.concatenate([w1[d:2 * d]] * N, axis=0)
    ss = (x0p * rwexp[:, None, :]).reshape(BB * N, Nd)
    dd = (x0p * aexp).reshape(BB * N, Nd)
    p = (jnp.dot(ss, tw1s, preferred_element_type=f32)
         + jnp.dot(dd, tw1d, preferred_element_type=f32)
         + b1_ref[...])

    yp = jnp.sum(x0p, axis=1) * (1.0 / N)
    y3 = jnp.concatenate(
        [yp[:, None, k * d:(k + 1) * d] for k in range(N)], axis=1)
    nctx = jax.lax.dot_general(a, y3, (((1,), (1,)), ((0,), (0,))),
                               preferred_element_type=f32)
    q = jnp.dot(nctx.reshape(BB * N, d), w1[2 * d:3 * d],
                preferred_element_type=f32)

    p3 = p.reshape(BB, N, d)
    ptile = jnp.concatenate([p3] * N, axis=2)
    q3 = q.reshape(BB, N, d)
    qpack = jnp.concatenate(
        [q3[:, j:j + 1, :] for j in range(N)], axis=2)
    out_ref[...] = ptile + qpack


def kernel(A, X, W0, b0, W1, b1):
    B, N, _, d = X.shape
    outdim = W1.shape[1]
    BB = 32
    grid = (B // BB,)
    out = pl.pallas_call(
        _fused_kernel,
        grid=grid,
        in_specs=[
            pl.BlockSpec((BB, N, N), lambda b: (b, 0, 0)),
            pl.BlockSpec((BB, N, N * d), lambda b: (b, 0, 0)),
            pl.BlockSpec((d, d), lambda b: (0, 0)),
            pl.BlockSpec((1, d), lambda b: (0, 0)),
            pl.BlockSpec((3 * d, outdim), lambda b: (0, 0)),
            pl.BlockSpec((1, outdim), lambda b: (0, 0)),
        ],
        out_specs=pl.BlockSpec((BB, N, N * outdim), lambda b: (b, 0, 0)),
        out_shape=jax.ShapeDtypeStruct((B, N, N * outdim), jnp.float32),
        compiler_params=pltpu.CompilerParams(
            dimension_semantics=("parallel",)),
    )(A, X.reshape(B, N, N * d), W0, b0.reshape(1, d), W1,
      b1.reshape(1, outdim))
    return out.reshape(B, N, N, outdim)
